# Initial kernel scaffold; baseline (speedup 1.0000x reference)
import functools

import jax
import jax.numpy as jnp
from jax import lax
from jax.experimental import pallas as pl
from jax.experimental.pallas import tpu as pltpu, tpu_sc as plsc

N = 10000
E = 160000
D = 128
CHUNK = 80
CHUNKS_PER_TILE = E // (16 * CHUNK)
ROWS_PER_TILE = N // 16
F32 = jnp.float32


def _zero_phase(sid, zer_acc, zer_cnt, acc, cntacc, has_cnt):
    r0 = sid * ROWS_PER_TILE
    pltpu.sync_copy(zer_acc, acc.at[pl.ds(r0, ROWS_PER_TILE)])
    if has_cnt:
        pltpu.sync_copy(zer_cnt, cntacc.at[pl.ds(r0, ROWS_PER_TILE)])


def _accum_phase(sid, ztab, ei, acc, cntacc, has_cnt, eidx_v, rows_v, ones_v, sem):
    def body(j, carry):
        base = sid * (E // 16) + j * CHUNK
        pltpu.sync_copy(ei.at[:, pl.ds(base, CHUNK)], eidx_v)
        pltpu.async_copy(ztab.at[eidx_v.at[0]], rows_v, sem).wait()
        pltpu.sync_copy(rows_v, acc.at[eidx_v.at[1]], add=True)
        if has_cnt:
            pltpu.sync_copy(ones_v, cntacc.at[eidx_v.at[1]], add=True)
        return carry

    lax.fori_loop(0, CHUNKS_PER_TILE, body, 0)


def _flush_phase(sid, acc, cntacc, aggout, b, cntout, has_cnt):
    r0 = sid * ROWS_PER_TILE
    pltpu.sync_copy(acc.at[pl.ds(r0, ROWS_PER_TILE)],
                    aggout.at[b, pl.ds(r0, ROWS_PER_TILE)])
    if has_cnt:
        pltpu.sync_copy(cntacc.at[pl.ds(r0, ROWS_PER_TILE)],
                        cntout.at[pl.ds(r0, ROWS_PER_TILE)])


@functools.partial(
    pl.kernel,
    out_type=[jax.ShapeDtypeStruct((2, N, D), F32)] * 4
    + [jax.ShapeDtypeStruct((N, 16), F32)] * 4,
    mesh=plsc.VectorSubcoreMesh(core_axis_name="c", subcore_axis_name="s"),
    scratch_types=[
        pltpu.VMEM((2, CHUNK), jnp.int32),
        pltpu.VMEM((CHUNK, D), F32),
        pltpu.VMEM((CHUNK, 16), F32),
        pltpu.VMEM_SHARED((N, D), F32),
        pltpu.VMEM_SHARED((N, 16), F32),
        pltpu.SemaphoreType.DMA,
    ],
)
def _sc_aggregate(z1b0, z1b1, z2b0, z2b1, ei_p, ei_s, ei_ct, ei_cf,
                  zer_acc, zer_cnt, ones_hbm,
                  agg_p, agg_cf, agg_s, agg_ct, cnt_p, cnt_cf, cnt_s, cnt_ct,
                  eidx_v, rows_v, ones_v, acc, cntacc, sem):
    core = lax.axis_index("c")
    sid = lax.axis_index("s")

    pltpu.sync_copy(ones_hbm, ones_v)

    combos0 = [(z1b0, ei_p, agg_p, 0, cnt_p), (z1b1, ei_p, agg_p, 1, None),
               (z2b0, ei_cf, agg_cf, 0, cnt_cf), (z2b1, ei_cf, agg_cf, 1, None)]
    combos1 = [(z2b0, ei_s, agg_s, 0, cnt_s), (z2b1, ei_s, agg_s, 1, None),
               (z1b0, ei_ct, agg_ct, 0, cnt_ct), (z1b1, ei_ct, agg_ct, 1, None)]

    for slot in range(4):
        zt0, ei0, out0, b0, cn0 = combos0[slot]
        zt1, ei1, out1, b1, cn1 = combos1[slot]

        @pl.when(core == 0)
        def _():
            _zero_phase(sid, zer_acc, zer_cnt, acc, cntacc, cn0 is not None)

        @pl.when(core == 1)
        def _():
            _zero_phase(sid, zer_acc, zer_cnt, acc, cntacc, cn1 is not None)

        plsc.subcore_barrier()

        @pl.when(core == 0)
        def _():
            _accum_phase(sid, zt0, ei0, acc, cntacc, cn0 is not None,
                         eidx_v, rows_v, ones_v, sem)

        @pl.when(core == 1)
        def _():
            _accum_phase(sid, zt1, ei1, acc, cntacc, cn1 is not None,
                         eidx_v, rows_v, ones_v, sem)

        plsc.subcore_barrier()

        @pl.when(core == 0)
        def _():
            _flush_phase(sid, acc, cntacc, out0, b0, cn0, cn0 is not None)

        @pl.when(core == 1)
        def _():
            _flush_phase(sid, acc, cntacc, out1, b1, cn1, cn1 is not None)

        plsc.subcore_barrier()


BLK = 500


def _tc_body(aggp_r, aggcf_r, aggs_r, aggct_r, cntp_r, cntcf_r, cnts_r, cntct_r,
             z1_r, z2_r, Wlp_r, Wlcf_r, Wls_r, Wlct_r, Wr1_r, Wr2_r,
             p1W_r, p2W_r, bc1_r, bc2_r, p1b_r, p2b_r,
             ln1g_r, ln1b_r, ln2g_r, ln2b_r, d1_r, d2_r):
    def dotT(x, w):
        return lax.dot_general(x, w, (((1,), (1,)), ((), ())),
                               preferred_element_type=F32)

    def layernorm(t, g, b):
        mu = jnp.mean(t, axis=1, keepdims=True)
        var = jnp.mean((t - mu) * (t - mu), axis=1, keepdims=True)
        return (t - mu) * lax.rsqrt(var + 1e-5) * g + b

    mp = aggp_r[0] * (1.0 / jnp.maximum(cntp_r[:, 0:1], 1.0))
    mcf = aggcf_r[0] * (1.0 / jnp.maximum(cntcf_r[:, 0:1], 1.0))
    o1 = dotT(mp, Wlp_r[...]) + dotT(mcf, Wlcf_r[...]) + dotT(z1_r[0], Wr1_r[...]) + bc1_r[...]
    t1 = dotT(o1, p1W_r[...]) + p1b_r[...]
    d1_r[0] = layernorm(t1, ln1g_r[...], ln1b_r[...])

    ms = aggs_r[0] * (1.0 / jnp.maximum(cnts_r[:, 0:1], 1.0))
    mct = aggct_r[0] * (1.0 / jnp.maximum(cntct_r[:, 0:1], 1.0))
    o2 = dotT(ms, Wls_r[...]) + dotT(mct, Wlct_r[...]) + dotT(z2_r[0], Wr2_r[...]) + bc2_r[...]
    t2 = dotT(o2, p2W_r[...]) + p2b_r[...]
    d2_r[0] = layernorm(t2, ln2g_r[...], ln2b_r[...])


def _tc_dense(agg_p, agg_cf, agg_s, agg_ct, cnt_p, cnt_cf, cnt_s, cnt_ct,
              z_1d, z_2d, Wl_pipe, Wl_cf, Wl_surface, Wl_ct, Wr1, Wr2,
              proj1_W, proj2_W, bc1, bc2, p1b, p2b, ln1g, ln1b, ln2g, ln2b):
    big = pl.BlockSpec((1, BLK, D), lambda b, i: (b, i, 0))
    cnt = pl.BlockSpec((BLK, 16), lambda b, i: (i, 0))
    mat = pl.BlockSpec((D, D), lambda b, i: (0, 0))
    vec = pl.BlockSpec((1, D), lambda b, i: (0, 0))
    return pl.pallas_call(
        _tc_body,
        grid=(2, N // BLK),
        in_specs=[big] * 4 + [cnt] * 4 + [big] * 2 + [mat] * 8 + [vec] * 8,
        out_specs=[big, big],
        out_shape=[jax.ShapeDtypeStruct((2, N, D), F32)] * 2,
    )(agg_p, agg_cf, agg_s, agg_ct, cnt_p, cnt_cf, cnt_s, cnt_ct,
      z_1d, z_2d, Wl_pipe, Wl_cf, Wl_surface, Wl_ct, Wr1, Wr2,
      proj1_W, proj2_W, bc1, bc2, p1b, p2b, ln1g, ln1b, ln2g, ln2b)


def kernel(z_1d, z_2d, edge_index_pipe, edge_index_surface,
           edge_index_couples_to, edge_index_couples_from,
           Wl_pipe, bl_pipe, Wr_pipe, Wl_surface, bl_surface, Wr_surface,
           Wl_ct, bl_ct, Wr_ct, Wl_cf, bl_cf, Wr_cf,
           proj1_W, proj1_b, proj2_W, proj2_b, ln1_g, ln1_b, ln2_g, ln2_b):
    zer_acc = jnp.zeros((ROWS_PER_TILE, D), F32)
    zer_cnt = jnp.zeros((ROWS_PER_TILE, 16), F32)
    ones_hbm = jnp.ones((CHUNK, 16), F32)

    (agg_p, agg_cf, agg_s, agg_ct,
     cnt_p, cnt_cf, cnt_s, cnt_ct) = _sc_aggregate(
        z_1d[0], z_1d[1], z_2d[0], z_2d[1],
        edge_index_pipe, edge_index_surface,
        edge_index_couples_to, edge_index_couples_from,
        zer_acc, zer_cnt, ones_hbm)

    d1, d2 = _tc_dense(
        agg_p, agg_cf, agg_s, agg_ct, cnt_p, cnt_cf, cnt_s, cnt_ct,
        z_1d, z_2d, Wl_pipe, Wl_cf, Wl_surface, Wl_ct,
        Wr_pipe + Wr_cf, Wr_surface + Wr_ct, proj1_W, proj2_W,
        (bl_pipe + bl_cf).reshape(1, D), (bl_surface + bl_ct).reshape(1, D),
        proj1_b.reshape(1, D), proj2_b.reshape(1, D),
        ln1_g.reshape(1, D), ln1_b.reshape(1, D),
        ln2_g.reshape(1, D), ln2_b.reshape(1, D))
    return (d1, d2)


# speedup vs baseline: 25.9768x; 25.9768x over previous
import functools

import jax
import jax.numpy as jnp
from jax import lax
from jax.experimental import pallas as pl
from jax.experimental.pallas import tpu as pltpu, tpu_sc as plsc

N = 10000
E = 160000
D = 128
CHUNK = 40
CHUNKS_PER_TILE = E // (16 * CHUNK)
FLUSH_ROWS = 1000
FLUSH_TILES = N // FLUSH_ROWS
STAGE = 40
HROWS = 80
F32 = jnp.float32


def _zero_phase(sid, zer_acc, stage_v, zidx_v, acc, iota):
    @pl.when(sid < FLUSH_TILES)
    def _():
        pltpu.sync_copy(zer_acc, stage_v)
        r0 = sid * FLUSH_ROWS

        def body(k, carry):
            pltpu.sync_copy(iota.at[pl.ds(r0 + k * STAGE, STAGE)], zidx_v)
            pltpu.sync_copy(stage_v, acc.at[zidx_v])
            return carry

        lax.fori_loop(0, FLUSH_ROWS // STAGE, body, 0)


def _accum_phase(sid, ztab, src, dst, acc, sidx_v, didx_v, rows_v, sem):
    def body(j, carry):
        base = sid * (E // 16) + j * CHUNK
        pltpu.sync_copy(src.at[pl.ds(base, CHUNK)], sidx_v)
        pltpu.sync_copy(dst.at[pl.ds(base, CHUNK)], didx_v)
        pltpu.async_copy(ztab.at[sidx_v], rows_v, sem).wait()
        pltpu.sync_copy(rows_v, acc.at[didx_v], add=True)
        return carry

    lax.fori_loop(0, CHUNKS_PER_TILE, body, 0)


def _count_phase(sid, dst, acc, didx_v, ones_v):
    def body(j, carry):
        base = sid * (E // 16) + j * CHUNK
        pltpu.sync_copy(dst.at[pl.ds(base, CHUNK)], didx_v)
        pltpu.sync_copy(ones_v, acc.at[didx_v], add=True)
        return carry

    lax.fori_loop(0, CHUNKS_PER_TILE, body, 0)


def _flush_phase(sid, acc, stage_v, zidx_v, zidx2_v, aggout, b, iota):
    @pl.when(sid < FLUSH_TILES)
    def _():
        r0 = sid * FLUSH_ROWS

        def body(k, carry):
            r = r0 + k * STAGE
            pltpu.sync_copy(iota.at[pl.ds(r, STAGE)], zidx_v)
            pltpu.sync_copy(iota.at[pl.ds(b * N + r, STAGE)], zidx2_v)
            pltpu.sync_copy(acc.at[zidx_v], stage_v)
            pltpu.sync_copy(stage_v, aggout.at[zidx2_v])
            return carry

        lax.fori_loop(0, FLUSH_ROWS // STAGE, body, 0)


@functools.lru_cache(maxsize=1)
def _build_sc_aggregate():
  kern = functools.partial(
    pl.kernel,
    out_type=[jax.ShapeDtypeStruct((2 * N, D), F32)] * 4
    + [jax.ShapeDtypeStruct((N, D), F32)] * 4,
    mesh=plsc.VectorSubcoreMesh(core_axis_name="c", subcore_axis_name="s",
                                num_cores=2, num_subcores=16),
    compiler_params=pltpu.CompilerParams(needs_layout_passes=False),
    scratch_types=[
        pltpu.VMEM((CHUNK,), jnp.int32),
        pltpu.VMEM((CHUNK,), jnp.int32),
        pltpu.VMEM((CHUNK, D), F32),
        pltpu.VMEM((CHUNK, D), F32),
        pltpu.VMEM((STAGE, D), F32),
        pltpu.VMEM((STAGE,), jnp.int32),
        pltpu.VMEM((STAGE,), jnp.int32),
        pltpu.VMEM_SHARED((N, D), F32),
        pltpu.SemaphoreType.DMA,
    ],
  )

  @kern
  def _sc_aggregate(z1b0, z1b1, z2b0, z2b1,
                    src_p, dst_p, src_s, dst_s, src_ct, dst_ct, src_cf, dst_cf,
                    zer_acc, ones_hbm, iota,
                    agg_p, agg_cf, agg_s, agg_ct, cnt_p, cnt_cf, cnt_s, cnt_ct,
                    sidx_v, didx_v, rows_v, ones_v, stage_v, zidx_v, zidx2_v,
                    acc, sem):
    pltpu.sync_copy(ones_hbm, ones_v)
    core = lax.axis_index("c")
    sid = lax.axis_index("s")

    combos0 = [(z1b0, src_p, dst_p, agg_p, 0),
               (z1b1, src_p, dst_p, agg_p, 1),
               (z2b0, src_cf, dst_cf, agg_cf, 0),
               (z2b1, src_cf, dst_cf, agg_cf, 1)]
    combos1 = [(z2b0, src_s, dst_s, agg_s, 0),
               (z2b1, src_s, dst_s, agg_s, 1),
               (z1b0, src_ct, dst_ct, agg_ct, 0),
               (z1b1, src_ct, dst_ct, agg_ct, 1)]

    for slot in range(4):
        zt0, s0, dd0, out0, b0 = combos0[slot]
        zt1, s1, dd1, out1, b1 = combos1[slot]

        @pl.when(core == 0)
        def _():
            _zero_phase(sid, zer_acc, stage_v, zidx_v, acc, iota)

        @pl.when(core == 1)
        def _():
            _zero_phase(sid, zer_acc, stage_v, zidx_v, acc, iota)

        plsc.subcore_barrier()

        @pl.when(core == 0)
        def _():
            _accum_phase(sid, zt0, s0, dd0, acc, sidx_v, didx_v, rows_v, sem)

        @pl.when(core == 1)
        def _():
            _accum_phase(sid, zt1, s1, dd1, acc, sidx_v, didx_v, rows_v, sem)

        plsc.subcore_barrier()

        @pl.when(core == 0)
        def _():
            _flush_phase(sid, acc, stage_v, zidx_v, zidx2_v, out0, b0, iota)

        @pl.when(core == 1)
        def _():
            _flush_phase(sid, acc, stage_v, zidx_v, zidx2_v, out1, b1, iota)

        plsc.subcore_barrier()

    cjobs0 = [(dst_p, cnt_p), (dst_cf, cnt_cf)]
    cjobs1 = [(dst_s, cnt_s), (dst_ct, cnt_ct)]
    for cj in range(2):
        dd0, co0 = cjobs0[cj]
        dd1, co1 = cjobs1[cj]

        @pl.when(core == 0)
        def _():
            _zero_phase(sid, zer_acc, stage_v, zidx_v, acc, iota)

        @pl.when(core == 1)
        def _():
            _zero_phase(sid, zer_acc, stage_v, zidx_v, acc, iota)

        plsc.subcore_barrier()

        @pl.when(core == 0)
        def _():
            _count_phase(sid, dd0, acc, didx_v, ones_v)

        @pl.when(core == 1)
        def _():
            _count_phase(sid, dd1, acc, didx_v, ones_v)

        plsc.subcore_barrier()

        @pl.when(core == 0)
        def _():
            _flush_phase(sid, acc, stage_v, zidx_v, zidx2_v, co0, 0, iota)

        @pl.when(core == 1)
        def _():
            _flush_phase(sid, acc, stage_v, zidx_v, zidx2_v, co1, 0, iota)

        plsc.subcore_barrier()

  return _sc_aggregate


BLK = 1000


def _tc_body(aggp_r, aggcf_r, aggs_r, aggct_r, cntp_r, cntcf_r, cnts_r, cntct_r,
             z1_r, z2_r, Wlp_r, Wlcf_r, Wls_r, Wlct_r, Wr1_r, Wr2_r,
             p1W_r, p2W_r, bc1_r, bc2_r, p1b_r, p2b_r,
             ln1g_r, ln1b_r, ln2g_r, ln2b_r, d1_r, d2_r):
    def dotT(x, w):
        return lax.dot_general(x, w, (((1,), (1,)), ((), ())),
                               preferred_element_type=F32)

    def layernorm(t, g, b):
        mu = jnp.mean(t, axis=1, keepdims=True)
        var = jnp.mean((t - mu) * (t - mu), axis=1, keepdims=True)
        return (t - mu) * lax.rsqrt(var + 1e-5) * g + b

    mp = aggp_r[0] * (1.0 / jnp.maximum(cntp_r[...], 1.0))
    mcf = aggcf_r[0] * (1.0 / jnp.maximum(cntcf_r[...], 1.0))
    o1 = dotT(mp, Wlp_r[...]) + dotT(mcf, Wlcf_r[...]) + dotT(z1_r[0], Wr1_r[...]) + bc1_r[...]
    t1 = dotT(o1, p1W_r[...]) + p1b_r[...]
    d1_r[0] = layernorm(t1, ln1g_r[...], ln1b_r[...])

    ms = aggs_r[0] * (1.0 / jnp.maximum(cnts_r[...], 1.0))
    mct = aggct_r[0] * (1.0 / jnp.maximum(cntct_r[...], 1.0))
    o2 = dotT(ms, Wls_r[...]) + dotT(mct, Wlct_r[...]) + dotT(z2_r[0], Wr2_r[...]) + bc2_r[...]
    t2 = dotT(o2, p2W_r[...]) + p2b_r[...]
    d2_r[0] = layernorm(t2, ln2g_r[...], ln2b_r[...])


def _tc_dense(agg_p, agg_cf, agg_s, agg_ct, cnt_p, cnt_cf, cnt_s, cnt_ct,
              z_1d, z_2d, Wl_pipe, Wl_cf, Wl_surface, Wl_ct, Wr1, Wr2,
              proj1_W, proj2_W, bc1, bc2, p1b, p2b, ln1g, ln1b, ln2g, ln2b):
    big = pl.BlockSpec((1, BLK, D), lambda b, i: (b, i, 0))
    cnt = pl.BlockSpec((BLK, 1), lambda b, i: (i, 0))
    mat = pl.BlockSpec((D, D), lambda b, i: (0, 0))
    vec = pl.BlockSpec((1, D), lambda b, i: (0, 0))
    return pl.pallas_call(
        _tc_body,
        grid=(2, N // BLK),
        in_specs=[big] * 4 + [cnt] * 4 + [big] * 2 + [mat] * 8 + [vec] * 8,
        out_specs=[big, big],
        out_shape=[jax.ShapeDtypeStruct((2, N, D), F32)] * 2,
    )(agg_p, agg_cf, agg_s, agg_ct, cnt_p, cnt_cf, cnt_s, cnt_ct,
      z_1d, z_2d, Wl_pipe, Wl_cf, Wl_surface, Wl_ct, Wr1, Wr2,
      proj1_W, proj2_W, bc1, bc2, p1b, p2b, ln1g, ln1b, ln2g, ln2b)


def kernel(z_1d, z_2d, edge_index_pipe, edge_index_surface,
           edge_index_couples_to, edge_index_couples_from,
           Wl_pipe, bl_pipe, Wr_pipe, Wl_surface, bl_surface, Wr_surface,
           Wl_ct, bl_ct, Wr_ct, Wl_cf, bl_cf, Wr_cf,
           proj1_W, proj1_b, proj2_W, proj2_b, ln1_g, ln1_b, ln2_g, ln2_b):
    zer_acc = jnp.zeros((STAGE, D), F32)
    ones_hbm = jnp.ones((CHUNK, D), F32)
    iota = jnp.arange(2 * N, dtype=jnp.int32)

    (agg_p, agg_cf, agg_s, agg_ct,
     cnt_p, cnt_cf, cnt_s, cnt_ct) = _build_sc_aggregate()(
        z_1d[0], z_1d[1], z_2d[0], z_2d[1],
        edge_index_pipe[0], edge_index_pipe[1],
        edge_index_surface[0], edge_index_surface[1],
        edge_index_couples_to[0], edge_index_couples_to[1],
        edge_index_couples_from[0], edge_index_couples_from[1],
        zer_acc, ones_hbm, iota)
    agg_p = agg_p.reshape(2, N, D)
    agg_cf = agg_cf.reshape(2, N, D)
    agg_s = agg_s.reshape(2, N, D)
    agg_ct = agg_ct.reshape(2, N, D)
    cnt_p = cnt_p[:, 0:1]
    cnt_cf = cnt_cf[:, 0:1]
    cnt_s = cnt_s[:, 0:1]
    cnt_ct = cnt_ct[:, 0:1]

    d1, d2 = _tc_dense(
        agg_p, agg_cf, agg_s, agg_ct, cnt_p, cnt_cf, cnt_s, cnt_ct,
        z_1d, z_2d, Wl_pipe, Wl_cf, Wl_surface, Wl_ct,
        Wr_pipe + Wr_cf, Wr_surface + Wr_ct, proj1_W, proj2_W,
        (bl_pipe + bl_cf).reshape(1, D), (bl_surface + bl_ct).reshape(1, D),
        proj1_b.reshape(1, D), proj2_b.reshape(1, D),
        ln1_g.reshape(1, D), ln1_b.reshape(1, D),
        ln2_g.reshape(1, D), ln2_b.reshape(1, D))
    return (d1, d2)


# speedup vs baseline: 40.1266x; 1.5447x over previous
import functools

import jax
import jax.numpy as jnp
from jax import lax
from jax.experimental import pallas as pl
from jax.experimental.pallas import tpu as pltpu, tpu_sc as plsc

N = 10000
E = 160000
D = 128
CHUNK = 40
CHUNKS_PER_TILE = E // (16 * CHUNK)
FLUSH_ROWS = 1000
FLUSH_TILES = N // FLUSH_ROWS
STAGE = 40
HROWS = 80
F32 = jnp.float32


def _zero_phase(sid, zer_acc, stage_v, zidx_v, acc, iota):
    @pl.when(sid < FLUSH_TILES)
    def _():
        pltpu.sync_copy(zer_acc, stage_v)
        r0 = sid * FLUSH_ROWS

        def body(k, carry):
            pltpu.sync_copy(iota.at[pl.ds(r0 + k * STAGE, STAGE)], zidx_v)
            pltpu.sync_copy(stage_v, acc.at[zidx_v])
            return carry

        lax.fori_loop(0, FLUSH_ROWS // STAGE, body, 0)


def _accum_phase(sid, ztab, src, dst, acc,
                 sidx2_v, didx2_v, rowsA_v, rowsB_v, gsemA, gsemB, sem):
    e0 = sid * (E // 16)
    rows = (rowsA_v, rowsB_v)
    gsems = (gsemA, gsemB)

    def load_and_fire(j, b):
        base = e0 + j * CHUNK
        pltpu.sync_copy(src.at[pl.ds(base, CHUNK)], sidx2_v.at[b])
        pltpu.sync_copy(dst.at[pl.ds(base, CHUNK)], didx2_v.at[b])
        return pltpu.async_copy(ztab.at[sidx2_v.at[b]], rows[b], gsems[b])

    load_and_fire(0, 0)

    def body(jj, carry):
        for b in range(2):
            j = jj * 2 + b
            nb = 1 - b

            @pl.when(j + 1 < CHUNKS_PER_TILE)
            def _():
                load_and_fire(j + 1, nb)

            pltpu.make_async_copy(ztab.at[sidx2_v.at[b]], rows[b],
                                  gsems[b]).wait()
            pltpu.sync_copy(rows[b], acc.at[didx2_v.at[b]], add=True)
        return carry

    lax.fori_loop(0, (CHUNKS_PER_TILE + 1) // 2, body, 0)


def _unused(sem):
    del sem


def _count_phase(sid, dst, acc, didx2_v, ones_v, gsemA, gsemB):
    e0 = sid * (E // 16)
    sems = (gsemA, gsemB)

    def body(jj, carry):
        for b in range(2):
            j = jj * 2 + b

            @pl.when(j < CHUNKS_PER_TILE)
            def _():
                @pl.when(j >= 2)
                def _():
                    pltpu.make_async_copy(ones_v, acc.at[didx2_v.at[b]],
                                          sems[b]).wait()
                base = e0 + j * CHUNK
                pltpu.sync_copy(dst.at[pl.ds(base, CHUNK)], didx2_v.at[b])
                pltpu.async_copy(ones_v, acc.at[didx2_v.at[b]], sems[b],
                                 add=True)
        return carry

    lax.fori_loop(0, (CHUNKS_PER_TILE + 1) // 2, body, 0)
    for b in range(2):
        pltpu.make_async_copy(ones_v, acc.at[didx2_v.at[b]], sems[b]).wait()


def _flush_phase(sid, acc, stage_v, zidx_v, zidx2_v, aggout, b, iota):
    @pl.when(sid < FLUSH_TILES)
    def _():
        r0 = sid * FLUSH_ROWS

        def body(k, carry):
            r = r0 + k * STAGE
            pltpu.sync_copy(iota.at[pl.ds(r, STAGE)], zidx_v)
            pltpu.sync_copy(iota.at[pl.ds(b * N + r, STAGE)], zidx2_v)
            pltpu.sync_copy(acc.at[zidx_v], stage_v)
            pltpu.sync_copy(stage_v, aggout.at[zidx2_v])
            return carry

        lax.fori_loop(0, FLUSH_ROWS // STAGE, body, 0)


@functools.lru_cache(maxsize=1)
def _build_sc_aggregate():
  kern = functools.partial(
    pl.kernel,
    out_type=[jax.ShapeDtypeStruct((2 * N, D), F32)] * 4
    + [jax.ShapeDtypeStruct((N, D), F32)] * 4,
    mesh=plsc.VectorSubcoreMesh(core_axis_name="c", subcore_axis_name="s",
                                num_cores=2, num_subcores=16),
    compiler_params=pltpu.CompilerParams(needs_layout_passes=False),
    scratch_types=[
        pltpu.VMEM((2, CHUNK), jnp.int32),
        pltpu.VMEM((2, CHUNK), jnp.int32),
        pltpu.VMEM((CHUNK, D), F32),
        pltpu.VMEM((CHUNK, D), F32),
        pltpu.VMEM((CHUNK, D), F32),
        pltpu.VMEM((STAGE, D), F32),
        pltpu.VMEM((STAGE,), jnp.int32),
        pltpu.VMEM((STAGE,), jnp.int32),
        pltpu.VMEM_SHARED((N, D), F32),
        pltpu.SemaphoreType.DMA,
        pltpu.SemaphoreType.DMA,
        pltpu.SemaphoreType.DMA,
    ],
  )

  @kern
  def _sc_aggregate(z1b0, z1b1, z2b0, z2b1,
                    src_p, dst_p, src_s, dst_s, src_ct, dst_ct, src_cf, dst_cf,
                    zer_acc, ones_hbm, iota,
                    agg_p, agg_cf, agg_s, agg_ct, cnt_p, cnt_cf, cnt_s, cnt_ct,
                    sidx2_v, didx2_v, rowsA_v, rowsB_v, ones_v, stage_v,
                    zidx_v, zidx2_v, acc, sem, gsemA, gsemB):
    pltpu.sync_copy(ones_hbm, ones_v)
    core = lax.axis_index("c")
    sid = lax.axis_index("s")

    combos0 = [(z1b0, src_p, dst_p, agg_p, 0),
               (z1b1, src_p, dst_p, agg_p, 1),
               (z2b0, src_cf, dst_cf, agg_cf, 0),
               (z2b1, src_cf, dst_cf, agg_cf, 1)]
    combos1 = [(z2b0, src_s, dst_s, agg_s, 0),
               (z2b1, src_s, dst_s, agg_s, 1),
               (z1b0, src_ct, dst_ct, agg_ct, 0),
               (z1b1, src_ct, dst_ct, agg_ct, 1)]

    for slot in range(4):
        zt0, s0, dd0, out0, b0 = combos0[slot]
        zt1, s1, dd1, out1, b1 = combos1[slot]

        @pl.when(core == 0)
        def _():
            _zero_phase(sid, zer_acc, stage_v, zidx_v, acc, iota)

        @pl.when(core == 1)
        def _():
            _zero_phase(sid, zer_acc, stage_v, zidx_v, acc, iota)

        plsc.subcore_barrier()

        @pl.when(core == 0)
        def _():
            _accum_phase(sid, zt0, s0, dd0, acc, sidx2_v, didx2_v,
                         rowsA_v, rowsB_v, gsemA, gsemB, sem)

        @pl.when(core == 1)
        def _():
            _accum_phase(sid, zt1, s1, dd1, acc, sidx2_v, didx2_v,
                         rowsA_v, rowsB_v, gsemA, gsemB, sem)

        plsc.subcore_barrier()

        @pl.when(core == 0)
        def _():
            _flush_phase(sid, acc, stage_v, zidx_v, zidx2_v, out0, b0, iota)

        @pl.when(core == 1)
        def _():
            _flush_phase(sid, acc, stage_v, zidx_v, zidx2_v, out1, b1, iota)

        plsc.subcore_barrier()

    cjobs0 = [(dst_p, cnt_p), (dst_cf, cnt_cf)]
    cjobs1 = [(dst_s, cnt_s), (dst_ct, cnt_ct)]
    for cj in range(2):
        dd0, co0 = cjobs0[cj]
        dd1, co1 = cjobs1[cj]

        @pl.when(core == 0)
        def _():
            _zero_phase(sid, zer_acc, stage_v, zidx_v, acc, iota)

        @pl.when(core == 1)
        def _():
            _zero_phase(sid, zer_acc, stage_v, zidx_v, acc, iota)

        plsc.subcore_barrier()

        @pl.when(core == 0)
        def _():
            _count_phase(sid, dd0, acc, didx2_v, ones_v, gsemA, gsemB)

        @pl.when(core == 1)
        def _():
            _count_phase(sid, dd1, acc, didx2_v, ones_v, gsemA, gsemB)

        plsc.subcore_barrier()

        @pl.when(core == 0)
        def _():
            _flush_phase(sid, acc, stage_v, zidx_v, zidx2_v, co0, 0, iota)

        @pl.when(core == 1)
        def _():
            _flush_phase(sid, acc, stage_v, zidx_v, zidx2_v, co1, 0, iota)

        plsc.subcore_barrier()

  return _sc_aggregate


BLK = 1000


def _tc_body(aggp_r, aggcf_r, aggs_r, aggct_r, cntp_r, cntcf_r, cnts_r, cntct_r,
             z1_r, z2_r, Wlp_r, Wlcf_r, Wls_r, Wlct_r, Wr1_r, Wr2_r,
             p1W_r, p2W_r, bc1_r, bc2_r, p1b_r, p2b_r,
             ln1g_r, ln1b_r, ln2g_r, ln2b_r, d1_r, d2_r):
    def dotT(x, w):
        return lax.dot_general(x, w, (((1,), (1,)), ((), ())),
                               preferred_element_type=F32)

    def layernorm(t, g, b):
        mu = jnp.mean(t, axis=1, keepdims=True)
        var = jnp.mean((t - mu) * (t - mu), axis=1, keepdims=True)
        return (t - mu) * lax.rsqrt(var + 1e-5) * g + b

    mp = aggp_r[0] * (1.0 / jnp.maximum(cntp_r[...], 1.0))
    mcf = aggcf_r[0] * (1.0 / jnp.maximum(cntcf_r[...], 1.0))
    o1 = dotT(mp, Wlp_r[...]) + dotT(mcf, Wlcf_r[...]) + dotT(z1_r[0], Wr1_r[...]) + bc1_r[...]
    t1 = dotT(o1, p1W_r[...]) + p1b_r[...]
    d1_r[0] = layernorm(t1, ln1g_r[...], ln1b_r[...])

    ms = aggs_r[0] * (1.0 / jnp.maximum(cnts_r[...], 1.0))
    mct = aggct_r[0] * (1.0 / jnp.maximum(cntct_r[...], 1.0))
    o2 = dotT(ms, Wls_r[...]) + dotT(mct, Wlct_r[...]) + dotT(z2_r[0], Wr2_r[...]) + bc2_r[...]
    t2 = dotT(o2, p2W_r[...]) + p2b_r[...]
    d2_r[0] = layernorm(t2, ln2g_r[...], ln2b_r[...])


def _tc_dense(agg_p, agg_cf, agg_s, agg_ct, cnt_p, cnt_cf, cnt_s, cnt_ct,
              z_1d, z_2d, Wl_pipe, Wl_cf, Wl_surface, Wl_ct, Wr1, Wr2,
              proj1_W, proj2_W, bc1, bc2, p1b, p2b, ln1g, ln1b, ln2g, ln2b):
    big = pl.BlockSpec((1, BLK, D), lambda b, i: (b, i, 0))
    cnt = pl.BlockSpec((BLK, 1), lambda b, i: (i, 0))
    mat = pl.BlockSpec((D, D), lambda b, i: (0, 0))
    vec = pl.BlockSpec((1, D), lambda b, i: (0, 0))
    return pl.pallas_call(
        _tc_body,
        grid=(2, N // BLK),
        in_specs=[big] * 4 + [cnt] * 4 + [big] * 2 + [mat] * 8 + [vec] * 8,
        out_specs=[big, big],
        out_shape=[jax.ShapeDtypeStruct((2, N, D), F32)] * 2,
    )(agg_p, agg_cf, agg_s, agg_ct, cnt_p, cnt_cf, cnt_s, cnt_ct,
      z_1d, z_2d, Wl_pipe, Wl_cf, Wl_surface, Wl_ct, Wr1, Wr2,
      proj1_W, proj2_W, bc1, bc2, p1b, p2b, ln1g, ln1b, ln2g, ln2b)


def kernel(z_1d, z_2d, edge_index_pipe, edge_index_surface,
           edge_index_couples_to, edge_index_couples_from,
           Wl_pipe, bl_pipe, Wr_pipe, Wl_surface, bl_surface, Wr_surface,
           Wl_ct, bl_ct, Wr_ct, Wl_cf, bl_cf, Wr_cf,
           proj1_W, proj1_b, proj2_W, proj2_b, ln1_g, ln1_b, ln2_g, ln2_b):
    zer_acc = jnp.zeros((STAGE, D), F32)
    ones_hbm = jnp.ones((CHUNK, D), F32)
    iota = jnp.arange(2 * N, dtype=jnp.int32)

    (agg_p, agg_cf, agg_s, agg_ct,
     cnt_p, cnt_cf, cnt_s, cnt_ct) = _build_sc_aggregate()(
        z_1d[0], z_1d[1], z_2d[0], z_2d[1],
        edge_index_pipe[0], edge_index_pipe[1],
        edge_index_surface[0], edge_index_surface[1],
        edge_index_couples_to[0], edge_index_couples_to[1],
        edge_index_couples_from[0], edge_index_couples_from[1],
        zer_acc, ones_hbm, iota)
    agg_p = agg_p.reshape(2, N, D)
    agg_cf = agg_cf.reshape(2, N, D)
    agg_s = agg_s.reshape(2, N, D)
    agg_ct = agg_ct.reshape(2, N, D)
    cnt_p = cnt_p[:, 0:1]
    cnt_cf = cnt_cf[:, 0:1]
    cnt_s = cnt_s[:, 0:1]
    cnt_ct = cnt_ct[:, 0:1]

    d1, d2 = _tc_dense(
        agg_p, agg_cf, agg_s, agg_ct, cnt_p, cnt_cf, cnt_s, cnt_ct,
        z_1d, z_2d, Wl_pipe, Wl_cf, Wl_surface, Wl_ct,
        Wr_pipe + Wr_cf, Wr_surface + Wr_ct, proj1_W, proj2_W,
        (bl_pipe + bl_cf).reshape(1, D), (bl_surface + bl_ct).reshape(1, D),
        proj1_b.reshape(1, D), proj2_b.reshape(1, D),
        ln1_g.reshape(1, D), ln1_b.reshape(1, D),
        ln2_g.reshape(1, D), ln2_b.reshape(1, D))
    return (d1, d2)


# speedup vs baseline: 41.7105x; 1.0395x over previous
import functools

import jax
import jax.numpy as jnp
from jax import lax
from jax.experimental import pallas as pl
from jax.experimental.pallas import tpu as pltpu, tpu_sc as plsc

N = 10000
E = 160000
D = 128
CHUNK = 40
CHUNKS_PER_TILE = E // (16 * CHUNK)
FLUSH_ROWS = 1000
FLUSH_TILES = N // FLUSH_ROWS
STAGE = 40
HROWS = 80
F32 = jnp.float32


def _zero_phase(sid, zer_acc, stage_v, zidx_v, zidx2_v, zsems, acc, iota):
    @pl.when(sid < FLUSH_TILES)
    def _():
        pltpu.sync_copy(zer_acc, stage_v)
        r0 = sid * FLUSH_ROWS
        zbufs = (zidx_v, zidx2_v)

        def body(kk, carry):
            for b in range(2):
                k = kk * 2 + b

                @pl.when(k < FLUSH_ROWS // STAGE)
                def _():
                    @pl.when(k >= 2)
                    def _():
                        pltpu.make_async_copy(stage_v, acc.at[zbufs[b]],
                                              zsems[b]).wait()
                    pltpu.sync_copy(iota.at[pl.ds(r0 + k * STAGE, STAGE)],
                                    zbufs[b])
                    pltpu.async_copy(stage_v, acc.at[zbufs[b]], zsems[b])
            return carry

        lax.fori_loop(0, (FLUSH_ROWS // STAGE + 1) // 2, body, 0)
        for b in range(2):
            pltpu.make_async_copy(stage_v, acc.at[zbufs[b]], zsems[b]).wait()


def _accum_phase(sid, ztab, src, dst, acc,
                 sidx2_v, didx2_v, rowsA_v, rowsB_v, gsemA, gsemB, sem):
    e0 = sid * (E // 16)
    rows = (rowsA_v, rowsB_v)
    gsems = (gsemA, gsemB)

    def load_and_fire(j, b):
        base = e0 + j * CHUNK
        pltpu.sync_copy(src.at[pl.ds(base, CHUNK)], sidx2_v.at[b])
        pltpu.sync_copy(dst.at[pl.ds(base, CHUNK)], didx2_v.at[b])
        return pltpu.async_copy(ztab.at[sidx2_v.at[b]], rows[b], gsems[b])

    load_and_fire(0, 0)

    def body(jj, carry):
        for b in range(2):
            j = jj * 2 + b
            nb = 1 - b

            @pl.when(j + 1 < CHUNKS_PER_TILE)
            def _():
                load_and_fire(j + 1, nb)

            pltpu.make_async_copy(ztab.at[sidx2_v.at[b]], rows[b],
                                  gsems[b]).wait()
            pltpu.sync_copy(rows[b], acc.at[didx2_v.at[b]], add=True)
        return carry

    lax.fori_loop(0, (CHUNKS_PER_TILE + 1) // 2, body, 0)


def _count_phase(sid, dst, acc, didx2_v, ones_v, gsemA, gsemB):
    e0 = sid * (E // 16)
    sems = (gsemA, gsemB)

    def body(jj, carry):
        for b in range(2):
            j = jj * 2 + b

            @pl.when(j < CHUNKS_PER_TILE)
            def _():
                @pl.when(j >= 2)
                def _():
                    pltpu.make_async_copy(ones_v, acc.at[didx2_v.at[b]],
                                          sems[b]).wait()
                base = e0 + j * CHUNK
                pltpu.sync_copy(dst.at[pl.ds(base, CHUNK)], didx2_v.at[b])
                pltpu.async_copy(ones_v, acc.at[didx2_v.at[b]], sems[b],
                                 add=True)
        return carry

    lax.fori_loop(0, (CHUNKS_PER_TILE + 1) // 2, body, 0)
    for b in range(2):
        pltpu.make_async_copy(ones_v, acc.at[didx2_v.at[b]], sems[b]).wait()


def _flush_phase(sid, acc, sidx2_v, didx2_v, rowsA_v, rowsB_v, wsems,
                 aggout, b, iota):
    @pl.when(sid < FLUSH_TILES)
    def _():
        r0 = sid * FLUSH_ROWS
        rows = (rowsA_v, rowsB_v)

        def body(kk, carry):
            for p in range(2):
                k = kk * 2 + p

                @pl.when(k < FLUSH_ROWS // STAGE)
                def _():
                    @pl.when(k >= 2)
                    def _():
                        pltpu.make_async_copy(rows[p],
                                              aggout.at[didx2_v.at[p]],
                                              wsems[p]).wait()
                    r = r0 + k * STAGE
                    pltpu.sync_copy(iota.at[pl.ds(r, STAGE)], sidx2_v.at[p])
                    pltpu.sync_copy(iota.at[pl.ds(b * N + r, STAGE)],
                                    didx2_v.at[p])
                    pltpu.sync_copy(acc.at[sidx2_v.at[p]], rows[p])
                    pltpu.async_copy(rows[p], aggout.at[didx2_v.at[p]],
                                     wsems[p])
            return carry

        lax.fori_loop(0, (FLUSH_ROWS // STAGE + 1) // 2, body, 0)
        for p in range(2):
            pltpu.make_async_copy(rows[p], aggout.at[didx2_v.at[p]],
                                  wsems[p]).wait()


@functools.lru_cache(maxsize=1)
def _build_sc_aggregate():
  kern = functools.partial(
    pl.kernel,
    out_type=[jax.ShapeDtypeStruct((2 * N, D), F32)] * 4
    + [jax.ShapeDtypeStruct((N, D), F32)] * 4,
    mesh=plsc.VectorSubcoreMesh(core_axis_name="c", subcore_axis_name="s",
                                num_cores=2, num_subcores=16),
    compiler_params=pltpu.CompilerParams(needs_layout_passes=False),
    scratch_types=[
        pltpu.VMEM((2, CHUNK), jnp.int32),
        pltpu.VMEM((2, CHUNK), jnp.int32),
        pltpu.VMEM((CHUNK, D), F32),
        pltpu.VMEM((CHUNK, D), F32),
        pltpu.VMEM((CHUNK, D), F32),
        pltpu.VMEM((STAGE, D), F32),
        pltpu.VMEM((STAGE,), jnp.int32),
        pltpu.VMEM((STAGE,), jnp.int32),
        pltpu.VMEM_SHARED((N, D), F32),
        pltpu.SemaphoreType.DMA,
        pltpu.SemaphoreType.DMA,
        pltpu.SemaphoreType.DMA,
    ],
  )

  @kern
  def _sc_aggregate(z1b0, z1b1, z2b0, z2b1,
                    src_p, dst_p, src_s, dst_s, src_ct, dst_ct, src_cf, dst_cf,
                    zer_acc, ones_hbm, iota,
                    agg_p, agg_cf, agg_s, agg_ct, cnt_p, cnt_cf, cnt_s, cnt_ct,
                    sidx2_v, didx2_v, rowsA_v, rowsB_v, ones_v, stage_v,
                    zidx_v, zidx2_v, acc, sem, gsemA, gsemB):
    pltpu.sync_copy(ones_hbm, ones_v)
    core = lax.axis_index("c")
    sid = lax.axis_index("s")

    combos0 = [(z1b0, src_p, dst_p, agg_p, 0),
               (z1b1, src_p, dst_p, agg_p, 1),
               (z2b0, src_cf, dst_cf, agg_cf, 0),
               (z2b1, src_cf, dst_cf, agg_cf, 1)]
    combos1 = [(z2b0, src_s, dst_s, agg_s, 0),
               (z2b1, src_s, dst_s, agg_s, 1),
               (z1b0, src_ct, dst_ct, agg_ct, 0),
               (z1b1, src_ct, dst_ct, agg_ct, 1)]

    for slot in range(4):
        zt0, s0, dd0, out0, b0 = combos0[slot]
        zt1, s1, dd1, out1, b1 = combos1[slot]

        @pl.when(core == 0)
        def _():
            _zero_phase(sid, zer_acc, stage_v, zidx_v, zidx2_v,
                        (gsemA, gsemB), acc, iota)

        @pl.when(core == 1)
        def _():
            _zero_phase(sid, zer_acc, stage_v, zidx_v, zidx2_v,
                        (gsemA, gsemB), acc, iota)

        plsc.subcore_barrier()

        @pl.when(core == 0)
        def _():
            _accum_phase(sid, zt0, s0, dd0, acc, sidx2_v, didx2_v,
                         rowsA_v, rowsB_v, gsemA, gsemB, sem)

        @pl.when(core == 1)
        def _():
            _accum_phase(sid, zt1, s1, dd1, acc, sidx2_v, didx2_v,
                         rowsA_v, rowsB_v, gsemA, gsemB, sem)

        plsc.subcore_barrier()

        @pl.when(core == 0)
        def _():
            _flush_phase(sid, acc, sidx2_v, didx2_v, rowsA_v, rowsB_v,
                         (gsemA, gsemB), out0, b0, iota)

        @pl.when(core == 1)
        def _():
            _flush_phase(sid, acc, sidx2_v, didx2_v, rowsA_v, rowsB_v,
                         (gsemA, gsemB), out1, b1, iota)

        plsc.subcore_barrier()

    cjobs0 = [(dst_p, cnt_p), (dst_cf, cnt_cf)]
    cjobs1 = [(dst_s, cnt_s), (dst_ct, cnt_ct)]
    for cj in range(2):
        dd0, co0 = cjobs0[cj]
        dd1, co1 = cjobs1[cj]

        @pl.when(core == 0)
        def _():
            _zero_phase(sid, zer_acc, stage_v, zidx_v, zidx2_v,
                        (gsemA, gsemB), acc, iota)

        @pl.when(core == 1)
        def _():
            _zero_phase(sid, zer_acc, stage_v, zidx_v, zidx2_v,
                        (gsemA, gsemB), acc, iota)

        plsc.subcore_barrier()

        @pl.when(core == 0)
        def _():
            _count_phase(sid, dd0, acc, didx2_v, ones_v, gsemA, gsemB)

        @pl.when(core == 1)
        def _():
            _count_phase(sid, dd1, acc, didx2_v, ones_v, gsemA, gsemB)

        plsc.subcore_barrier()

        @pl.when(core == 0)
        def _():
            _flush_phase(sid, acc, sidx2_v, didx2_v, rowsA_v, rowsB_v,
                         (gsemA, gsemB), co0, 0, iota)

        @pl.when(core == 1)
        def _():
            _flush_phase(sid, acc, sidx2_v, didx2_v, rowsA_v, rowsB_v,
                         (gsemA, gsemB), co1, 0, iota)

        plsc.subcore_barrier()

  return _sc_aggregate


BLK = 1000


def _tc_body(aggp_r, aggcf_r, aggs_r, aggct_r, cntp_r, cntcf_r, cnts_r, cntct_r,
             z1_r, z2_r, Wlp_r, Wlcf_r, Wls_r, Wlct_r, Wr1_r, Wr2_r,
             p1W_r, p2W_r, bc1_r, bc2_r, p1b_r, p2b_r,
             ln1g_r, ln1b_r, ln2g_r, ln2b_r, d1_r, d2_r):
    def dotT(x, w):
        return lax.dot_general(x, w, (((1,), (1,)), ((), ())),
                               preferred_element_type=F32)

    def layernorm(t, g, b):
        mu = jnp.mean(t, axis=1, keepdims=True)
        var = jnp.mean((t - mu) * (t - mu), axis=1, keepdims=True)
        return (t - mu) * lax.rsqrt(var + 1e-5) * g + b

    mp = aggp_r[0] * (1.0 / jnp.maximum(cntp_r[...], 1.0))
    mcf = aggcf_r[0] * (1.0 / jnp.maximum(cntcf_r[...], 1.0))
    o1 = dotT(mp, Wlp_r[...]) + dotT(mcf, Wlcf_r[...]) + dotT(z1_r[0], Wr1_r[...]) + bc1_r[...]
    t1 = dotT(o1, p1W_r[...]) + p1b_r[...]
    d1_r[0] = layernorm(t1, ln1g_r[...], ln1b_r[...])

    ms = aggs_r[0] * (1.0 / jnp.maximum(cnts_r[...], 1.0))
    mct = aggct_r[0] * (1.0 / jnp.maximum(cntct_r[...], 1.0))
    o2 = dotT(ms, Wls_r[...]) + dotT(mct, Wlct_r[...]) + dotT(z2_r[0], Wr2_r[...]) + bc2_r[...]
    t2 = dotT(o2, p2W_r[...]) + p2b_r[...]
    d2_r[0] = layernorm(t2, ln2g_r[...], ln2b_r[...])


def _tc_dense(agg_p, agg_cf, agg_s, agg_ct, cnt_p, cnt_cf, cnt_s, cnt_ct,
              z_1d, z_2d, Wl_pipe, Wl_cf, Wl_surface, Wl_ct, Wr1, Wr2,
              proj1_W, proj2_W, bc1, bc2, p1b, p2b, ln1g, ln1b, ln2g, ln2b):
    big = pl.BlockSpec((1, BLK, D), lambda b, i: (b, i, 0))
    cnt = pl.BlockSpec((BLK, 1), lambda b, i: (i, 0))
    mat = pl.BlockSpec((D, D), lambda b, i: (0, 0))
    vec = pl.BlockSpec((1, D), lambda b, i: (0, 0))
    return pl.pallas_call(
        _tc_body,
        grid=(2, N // BLK),
        in_specs=[big] * 4 + [cnt] * 4 + [big] * 2 + [mat] * 8 + [vec] * 8,
        out_specs=[big, big],
        out_shape=[jax.ShapeDtypeStruct((2, N, D), F32)] * 2,
    )(agg_p, agg_cf, agg_s, agg_ct, cnt_p, cnt_cf, cnt_s, cnt_ct,
      z_1d, z_2d, Wl_pipe, Wl_cf, Wl_surface, Wl_ct, Wr1, Wr2,
      proj1_W, proj2_W, bc1, bc2, p1b, p2b, ln1g, ln1b, ln2g, ln2b)


def kernel(z_1d, z_2d, edge_index_pipe, edge_index_surface,
           edge_index_couples_to, edge_index_couples_from,
           Wl_pipe, bl_pipe, Wr_pipe, Wl_surface, bl_surface, Wr_surface,
           Wl_ct, bl_ct, Wr_ct, Wl_cf, bl_cf, Wr_cf,
           proj1_W, proj1_b, proj2_W, proj2_b, ln1_g, ln1_b, ln2_g, ln2_b):
    zer_acc = jnp.zeros((STAGE, D), F32)
    ones_hbm = jnp.ones((CHUNK, D), F32)
    iota = jnp.arange(2 * N, dtype=jnp.int32)

    (agg_p, agg_cf, agg_s, agg_ct,
     cnt_p, cnt_cf, cnt_s, cnt_ct) = _build_sc_aggregate()(
        z_1d[0], z_1d[1], z_2d[0], z_2d[1],
        edge_index_pipe[0], edge_index_pipe[1],
        edge_index_surface[0], edge_index_surface[1],
        edge_index_couples_to[0], edge_index_couples_to[1],
        edge_index_couples_from[0], edge_index_couples_from[1],
        zer_acc, ones_hbm, iota)
    agg_p = agg_p.reshape(2, N, D)
    agg_cf = agg_cf.reshape(2, N, D)
    agg_s = agg_s.reshape(2, N, D)
    agg_ct = agg_ct.reshape(2, N, D)
    cnt_p = cnt_p[:, 0:1]
    cnt_cf = cnt_cf[:, 0:1]
    cnt_s = cnt_s[:, 0:1]
    cnt_ct = cnt_ct[:, 0:1]

    d1, d2 = _tc_dense(
        agg_p, agg_cf, agg_s, agg_ct, cnt_p, cnt_cf, cnt_s, cnt_ct,
        z_1d, z_2d, Wl_pipe, Wl_cf, Wl_surface, Wl_ct,
        Wr_pipe + Wr_cf, Wr_surface + Wr_ct, proj1_W, proj2_W,
        (bl_pipe + bl_cf).reshape(1, D), (bl_surface + bl_ct).reshape(1, D),
        proj1_b.reshape(1, D), proj2_b.reshape(1, D),
        ln1_g.reshape(1, D), ln1_b.reshape(1, D),
        ln2_g.reshape(1, D), ln2_b.reshape(1, D))
    return (d1, d2)
